# Initial kernel scaffold; baseline (speedup 1.0000x reference)
import jax
import jax.numpy as jnp
from jax.experimental import pallas as pl


def kernel(x, W_gate):
    raise NotImplementedError("write your pallas kernel here")


# speedup vs baseline: 8.1970x; 8.1970x over previous
import functools

import jax
import jax.numpy as jnp
from jax import lax
from jax.experimental import pallas as pl
from jax.experimental.pallas import tpu as pltpu


def _matmul_body(w_ref, x_ref, out_ref):
    out_ref[...] = lax.dot_general(
        w_ref[...], x_ref[...],
        dimension_numbers=(((1,), (1,)), ((), ())),
        preferred_element_type=jnp.float32,
    )


def _select_body(logits_ref, expert_ref, weight_ref, *, k):
    logits = logits_ref[...]
    E, T = logits.shape

    b = lax.bitcast_convert_type(logits, jnp.int32)
    key = jnp.where(b >= 0, b, b ^ jnp.int32(0x7FFFFFFF))
    key = jnp.where(logits == 0.0, jnp.int32(0), key)

    def count_ge(thr):
        return jnp.sum((key >= thr).astype(jnp.int32), axis=1, keepdims=True)

    lo0 = jnp.min(key, axis=1, keepdims=True)
    hi0 = jnp.max(key, axis=1, keepdims=True)

    def bs_body(_, lohi):
        lo, hi = lohi
        mid = (lo & hi) + ((lo ^ hi) >> 1) + ((lo ^ hi) & 1)
        take = count_ge(mid) >= k
        return jnp.where(take, mid, lo), jnp.where(take, hi, mid - 1)

    thr, _ = lax.fori_loop(0, 32, bs_body, (lo0, hi0))

    cnt_gt = jnp.sum((key > thr).astype(jnp.int32), axis=1, keepdims=True)
    allowed = k - cnt_gt
    eq = key == thr
    tok = lax.broadcasted_iota(jnp.int32, (E, T), 1)

    def tie_body(_, lohi):
        lo, hi = lohi
        mid = (lo + hi) >> 1
        c = jnp.sum((eq & (tok <= mid)).astype(jnp.int32), axis=1, keepdims=True)
        good = c >= allowed
        return jnp.where(good, lo, mid + 1), jnp.where(good, mid, hi)

    cut, _ = lax.fori_loop(
        0, 13, tie_body,
        (jnp.zeros((E, 1), jnp.int32), jnp.full((E, 1), T - 1, jnp.int32)))

    sel = (key > thr) | (eq & (tok <= cut))

    eids = lax.broadcasted_iota(jnp.int32, (E, T), 0)
    e_star = jnp.max(jnp.where(sel, eids, -1), axis=0, keepdims=True)
    picked = e_star >= 0
    val = jnp.sum(jnp.where(eids == e_star, logits, 0.0), axis=0, keepdims=True)

    m = jnp.max(logits, axis=0, keepdims=True)
    denom = jnp.sum(jnp.exp(logits - m), axis=0, keepdims=True)
    w = jnp.exp(val - m) / denom

    expert_ref[...] = jnp.maximum(e_star, 0)
    weight_ref[...] = jnp.where(picked, w, jnp.float32(1.0))


def kernel(x, W_gate):
    B, S, D = x.shape
    E = W_gate.shape[0]
    T = B * S
    k = min(int(T / E * 1.0), T)

    xf = x.reshape(T, D)
    TBLK = 512
    logits = pl.pallas_call(
        _matmul_body,
        grid=(T // TBLK,),
        in_specs=[
            pl.BlockSpec((E, D), lambda i: (0, 0)),
            pl.BlockSpec((TBLK, D), lambda i: (i, 0)),
        ],
        out_specs=pl.BlockSpec((E, TBLK), lambda i: (0, i)),
        out_shape=jax.ShapeDtypeStruct((E, T), jnp.float32),
    )(W_gate, xf)

    expert, weight = pl.pallas_call(
        functools.partial(_select_body, k=k),
        out_shape=(
            jax.ShapeDtypeStruct((1, T), jnp.int32),
            jax.ShapeDtypeStruct((1, T), jnp.float32),
        ),
    )(logits)

    expert_out = expert.reshape(B, S, 1)
    weight_out = weight.reshape(B, S, 1).astype(x.dtype)
    aux_loss = jnp.array(0.0, dtype=x.dtype)
    return (expert_out, weight_out, aux_loss)
